# baseline (speedup 1.0000x reference)
import jax
import jax.numpy as jnp
from jax import lax
from jax.experimental import pallas as pl
from jax.experimental.pallas import tpu as pltpu

_B, _S, _C = 4, 8192, 768
_K = 8

_CH = 512
_NCH = _S // _CH
_GPC = _CH // _K
_NCAND = _K * _K
_TCAND = _NCH * _NCAND

_NEG_INF = float("-inf")
_I32_MAX = 2**31 - 1


def _body(x_ref, out_ref, cv_ref, ci_ref):
    j = pl.program_id(1)

    x2 = x_ref[0]
    x3 = x2.reshape(_GPC, _K, _C)
    l1c = jnp.max(x3, axis=1)

    g_iota = lax.broadcasted_iota(jnp.int32, (_GPC, _C), 0)
    sels = []
    for r in range(_K):
        m = jnp.max(l1c, axis=0, keepdims=True)
        sel = jnp.max(jnp.where(l1c == m, g_iota, -1), axis=0,
                      keepdims=True)
        sels.append(sel)
        if r < _K - 1:
            l1c = jnp.where(g_iota == sel, _NEG_INF, l1c)

    grp2 = lax.broadcasted_iota(jnp.int32, (_CH, _C), 0) // _K
    row8 = lax.broadcasted_iota(jnp.int32, (_K, _C), 0)
    vals = []
    idxs = []
    for r in range(_K):
        masked = jnp.where(grp2 == sels[r], x2, _NEG_INF)
        vals.append(jnp.max(masked.reshape(_GPC, _K, _C), axis=0))
        idxs.append((j * _GPC + sels[r]) * _K + row8)
    cv_ref[pl.ds(j * _NCAND, _NCAND), :] = jnp.concatenate(vals, axis=0)
    ci_ref[pl.ds(j * _NCAND, _NCAND), :] = jnp.concatenate(idxs, axis=0)

    @pl.when(j == _NCH - 1)
    def _finalize():
        cv = cv_ref[...]
        ci = ci_ref[...]

        kept_v = []
        kept_s = []
        for r in range(_K):
            m = jnp.max(cv, axis=0, keepdims=True)
            ps = jnp.max(jnp.where(cv == m, ci, -1), axis=0, keepdims=True)
            kept_v.append(m)
            kept_s.append(ps)
            if r < _K - 1:
                cv = jnp.where(ci == ps, _NEG_INF, cv)
        av = jnp.concatenate(kept_v, axis=0)
        ai = jnp.concatenate(kept_s, axis=0)

        outs = []
        for r in range(_K):
            mi = jnp.min(ai, axis=0, keepdims=True)
            outs.append(jnp.max(jnp.where(ai == mi, av, _NEG_INF), axis=0,
                                keepdims=True))
            if r < _K - 1:
                ai = jnp.where(ai == mi, _I32_MAX, ai)
        out_ref[0] = jnp.concatenate(outs, axis=0)


def kernel(x):
    return pl.pallas_call(
        _body,
        grid=(_B, _NCH),
        in_specs=[pl.BlockSpec((1, _CH, _C), lambda b, j: (b, j, 0))],
        out_specs=pl.BlockSpec((1, _K, _C), lambda b, j: (b, 0, 0)),
        out_shape=jax.ShapeDtypeStruct((_B, _K, _C), jnp.float32),
        scratch_shapes=[
            pltpu.VMEM((_TCAND, _C), jnp.float32),
            pltpu.VMEM((_TCAND, _C), jnp.int32),
        ],
    )(x)


# speedup vs baseline: 1.1571x; 1.1571x over previous
import jax
import jax.numpy as jnp
from jax import lax
from jax.experimental import pallas as pl
from jax.experimental.pallas import tpu as pltpu

_B, _S, _C = 4, 8192, 768
_K = 8

_CH = 1024
_NCH = _S // _CH
_GPC = _CH // _K
_NCAND = _K * _K
_TCAND = _NCH * _NCAND

_NEG_INF = float("-inf")
_I32_MAX = 2**31 - 1


def _body(x_ref, out_ref, cv_ref, ci_ref):
    j = pl.program_id(1)

    x2 = x_ref[0]
    x3 = x2.reshape(_GPC, _K, _C)
    l1c = jnp.max(x3, axis=1)

    g_iota = lax.broadcasted_iota(jnp.int32, (_GPC, _C), 0)
    sels = []
    for r in range(_K):
        m = jnp.max(l1c, axis=0, keepdims=True)
        sel = jnp.max(jnp.where(l1c == m, g_iota, -1), axis=0,
                      keepdims=True)
        sels.append(sel)
        if r < _K - 1:
            l1c = jnp.where(g_iota == sel, _NEG_INF, l1c)

    grp2 = lax.broadcasted_iota(jnp.int32, (_CH, _C), 0) // _K
    row8 = lax.broadcasted_iota(jnp.int32, (_K, _C), 0)
    vals = []
    idxs = []
    for r in range(_K):
        masked = jnp.where(grp2 == sels[r], x2, _NEG_INF)
        vals.append(jnp.max(masked.reshape(_GPC, _K, _C), axis=0))
        idxs.append((j * _GPC + sels[r]) * _K + row8)
    cv_ref[pl.ds(j * _NCAND, _NCAND), :] = jnp.concatenate(vals, axis=0)
    ci_ref[pl.ds(j * _NCAND, _NCAND), :] = jnp.concatenate(idxs, axis=0)

    @pl.when(j == _NCH - 1)
    def _finalize():
        cv = cv_ref[...]
        ci = ci_ref[...]

        kept_v = []
        kept_s = []
        for r in range(_K):
            m = jnp.max(cv, axis=0, keepdims=True)
            ps = jnp.max(jnp.where(cv == m, ci, -1), axis=0, keepdims=True)
            kept_v.append(m)
            kept_s.append(ps)
            if r < _K - 1:
                cv = jnp.where(ci == ps, _NEG_INF, cv)
        av = jnp.concatenate(kept_v, axis=0)
        ai = jnp.concatenate(kept_s, axis=0)

        outs = []
        for r in range(_K):
            mi = jnp.min(ai, axis=0, keepdims=True)
            outs.append(jnp.max(jnp.where(ai == mi, av, _NEG_INF), axis=0,
                                keepdims=True))
            if r < _K - 1:
                ai = jnp.where(ai == mi, _I32_MAX, ai)
        out_ref[0] = jnp.concatenate(outs, axis=0)


def kernel(x):
    return pl.pallas_call(
        _body,
        grid=(_B, _NCH),
        in_specs=[pl.BlockSpec((1, _CH, _C), lambda b, j: (b, j, 0))],
        out_specs=pl.BlockSpec((1, _K, _C), lambda b, j: (b, 0, 0)),
        out_shape=jax.ShapeDtypeStruct((_B, _K, _C), jnp.float32),
        scratch_shapes=[
            pltpu.VMEM((_TCAND, _C), jnp.float32),
            pltpu.VMEM((_TCAND, _C), jnp.int32),
        ],
    )(x)
